# baseline (speedup 1.0000x reference)
import functools

import jax
import jax.numpy as jnp
import numpy as np
from jax.experimental import pallas as pl
from jax.experimental.pallas import tpu as pltpu


def _kimia_body(x_ref, xw_ref, wkvq_ref, wk2_ref,
                w1_ref, w2_ref, temb_ref, sins_ref, o_ref, k_scr, v_scr):
    f32 = jnp.float32
    NI = temb_ref.shape[0] - 1
    D = w1_ref.shape[0]
    DK = D
    NC = k_scr.shape[0]
    CB = k_scr.shape[2]

    biases = jnp.dot(sins_ref[...], wk2_ref[...], preferred_element_type=f32)

    A = [None] * NC
    for c in range(NC):
        kv0 = jnp.dot(x_ref[c * CB:(c + 1) * CB], xw_ref[...],
                      preferred_element_type=f32)
        k_scr[c, 0] = kv0[:, :DK]
        v_scr[c, 0] = kv0[:, DK:]
        A[c] = kv0[:, DK:]

    for t in range(NI - 1):
        for c in range(NC):
            h = jnp.dot(A[c], w1_ref[...],
                        preferred_element_type=f32) + temb_ref[t]
            Bt = jnp.dot(jax.nn.gelu(h), w2_ref[...],
                         preferred_element_type=f32)
            kvq = jnp.dot(Bt, wkvq_ref[...], preferred_element_type=f32)
            k_scr[c, t + 1] = kvq[:, :DK] + biases[t]
            v_scr[c, t + 1] = kvq[:, DK:DK + D]
            q = kvq[:, DK + D:]

            n = t + 2
            svals = [jnp.sum(q * k_scr[c, j], axis=-1, keepdims=True)
                     for j in range(n)]
            m = svals[0]
            for j in range(1, n):
                m = jnp.maximum(m, svals[j])
            evals = [jnp.exp(s - m) for s in svals]
            den = evals[0]
            for j in range(1, n):
                den = den + evals[j]
            r = 1.0 / den
            acc = (evals[0] * r) * v_scr[c, 0]
            for j in range(1, n):
                acc = acc + (evals[j] * r) * v_scr[c, j]
            A[c] = acc

    for c in range(NC):
        h = jnp.dot(A[c], w1_ref[...],
                    preferred_element_type=f32) + temb_ref[NI]
        o_ref[c * CB:(c + 1) * CB] = jnp.dot(
            jax.nn.gelu(h), w2_ref[...], preferred_element_type=f32)


@functools.partial(jax.jit, static_argnames=("interpret",))
def kernel(x, Wik, Wiv, Wq, Wk, Wv, W1, W2, t_emb, interpret=False):
    B, D = x.shape
    DK = Wq.shape[1]
    TR = Wk.shape[0] - D
    NI = t_emb.shape[0] - 1
    dt = x.dtype

    scale = np.float32(1.0 / np.sqrt(DK))
    wikp = jnp.pad(Wik, ((0, 0), (0, DK - Wik.shape[1])))
    xw = jnp.concatenate([wikp, Wiv], axis=1)
    wkvq = jnp.concatenate([Wk[:D], Wv, Wq * scale], axis=1)
    wk2 = Wk[D:]
    t_enc = jnp.pi * (0.5 ** jnp.arange(TR, dtype=dt))
    tvals = jnp.arange(NI, dtype=dt)
    sins = jnp.sin(tvals[:, None] * t_enc[None, :])

    BB = 256
    NC = 4
    CB = BB // NC
    grid = (B // BB,)
    full = lambda shape: pl.BlockSpec(shape, lambda i: tuple(0 for _ in shape))

    return pl.pallas_call(
        _kimia_body,
        out_shape=jax.ShapeDtypeStruct((B, D), dt),
        grid=grid,
        in_specs=[
            pl.BlockSpec((BB, D), lambda i: (i, 0)),
            full((D, DK + D)),
            full((D, DK + D + DK)),
            full((TR, DK)),
            full((D, D)),
            full((D, D)),
            full((NI + 1, D)),
            full((NI, TR)),
        ],
        out_specs=pl.BlockSpec((BB, D), lambda i: (i, 0)),
        scratch_shapes=[
            pltpu.VMEM((NC, NI, CB, DK), jnp.float32),
            pltpu.VMEM((NC, NI, CB, D), jnp.float32),
        ],
        compiler_params=pltpu.CompilerParams(
            dimension_semantics=("parallel",),
            vmem_limit_bytes=56 * 1024 * 1024,
        ),
        name="ind_kimia_fused",
        interpret=interpret,
    )(x, xw, wkvq, wk2, W1, W2, t_emb, sins)


# speedup vs baseline: 2.2877x; 2.2877x over previous
import functools

import jax
import jax.numpy as jnp
import numpy as np
from jax.experimental import pallas as pl
from jax.experimental.pallas import tpu as pltpu


def _kimia_body(x_ref, xw_ref, wkvq_ref, wk2_ref,
                w1_ref, w2_ref, temb_ref, sins_ref, o_ref, k_scr, v_scr):
    f32 = jnp.float32
    NI = temb_ref.shape[0] - 1
    D = w1_ref.shape[0]
    DK = D
    NC = k_scr.shape[0]
    CB = k_scr.shape[2]

    biases = jnp.dot(sins_ref[...], wk2_ref[...], preferred_element_type=f32)

    bf16 = jnp.bfloat16
    NL = DK // 128

    def slot_score(qb, kj):
        p = qb * kj
        s = p[:, :128]
        for l in range(1, NL):
            s = s + p[:, 128 * l:128 * (l + 1)]
        return jnp.sum(s, axis=-1, keepdims=True, dtype=f32)

    A = [None] * NC
    for c in range(NC):
        kv0 = jnp.dot(x_ref[c * CB:(c + 1) * CB], xw_ref[...],
                      preferred_element_type=f32)
        k_scr[c, 0] = kv0[:, :DK].astype(bf16)
        v_scr[c, 0] = kv0[:, DK:]
        A[c] = kv0[:, DK:]

    for t in range(NI - 1):
        for c in range(NC):
            h = jnp.dot(A[c], w1_ref[...],
                        preferred_element_type=f32) + temb_ref[t]
            Bt = jnp.dot(jax.nn.gelu(h), w2_ref[...],
                         preferred_element_type=f32)
            kvq = jnp.dot(Bt, wkvq_ref[...], preferred_element_type=f32)
            k_scr[c, t + 1] = (kvq[:, :DK] + biases[t]).astype(bf16)
            v_scr[c, t + 1] = kvq[:, DK:DK + D]
            qb = kvq[:, DK + D:].astype(bf16)

            n = t + 2
            svals = [slot_score(qb, k_scr[c, j]) for j in range(n)]
            m = svals[0]
            for j in range(1, n):
                m = jnp.maximum(m, svals[j])
            evals = [jnp.exp(s - m) for s in svals]
            den = evals[0]
            for j in range(1, n):
                den = den + evals[j]
            acc = evals[0] * v_scr[c, 0]
            for j in range(1, n):
                acc = acc + evals[j] * v_scr[c, j]
            A[c] = acc * (1.0 / den)

    for c in range(NC):
        h = jnp.dot(A[c], w1_ref[...],
                    preferred_element_type=f32) + temb_ref[NI]
        o_ref[c * CB:(c + 1) * CB] = jnp.dot(
            jax.nn.gelu(h), w2_ref[...], preferred_element_type=f32)


@functools.partial(jax.jit, static_argnames=("interpret",))
def kernel(x, Wik, Wiv, Wq, Wk, Wv, W1, W2, t_emb, interpret=False):
    B, D = x.shape
    DK = Wq.shape[1]
    TR = Wk.shape[0] - D
    NI = t_emb.shape[0] - 1
    dt = x.dtype

    scale = np.float32(1.0 / np.sqrt(DK))
    wikp = jnp.pad(Wik, ((0, 0), (0, DK - Wik.shape[1])))
    xw = jnp.concatenate([wikp, Wiv], axis=1)
    wkvq = jnp.concatenate([Wk[:D], Wv, Wq * scale], axis=1)
    wk2 = Wk[D:]
    t_enc = jnp.pi * (0.5 ** jnp.arange(TR, dtype=dt))
    tvals = jnp.arange(NI, dtype=dt)
    sins = jnp.sin(tvals[:, None] * t_enc[None, :])

    BB = 256
    NC = 1
    CB = BB // NC
    grid = (B // BB,)
    full = lambda shape: pl.BlockSpec(shape, lambda i: tuple(0 for _ in shape))

    return pl.pallas_call(
        _kimia_body,
        out_shape=jax.ShapeDtypeStruct((B, D), dt),
        grid=grid,
        in_specs=[
            pl.BlockSpec((BB, D), lambda i: (i, 0)),
            full((D, DK + D)),
            full((D, DK + D + DK)),
            full((TR, DK)),
            full((D, D)),
            full((D, D)),
            full((NI + 1, D)),
            full((NI, TR)),
        ],
        out_specs=pl.BlockSpec((BB, D), lambda i: (i, 0)),
        scratch_shapes=[
            pltpu.VMEM((NC, NI, CB, DK), jnp.bfloat16),
            pltpu.VMEM((NC, NI, CB, D), jnp.float32),
        ],
        compiler_params=pltpu.CompilerParams(
            dimension_semantics=("parallel",),
            vmem_limit_bytes=56 * 1024 * 1024,
        ),
        name="ind_kimia_fused",
        interpret=interpret,
    )(x, xw, wkvq, wk2, W1, W2, t_emb, sins)


# speedup vs baseline: 2.3227x; 1.0153x over previous
import functools

import jax
import jax.numpy as jnp
import numpy as np
from jax.experimental import pallas as pl
from jax.experimental.pallas import tpu as pltpu


def _kimia_body(x_ref, xw_ref, wkvq_ref, wk2_ref,
                w1_ref, w2_ref, temb_ref, sins_ref, o_ref, k_scr, v_scr):
    f32 = jnp.float32
    NI = temb_ref.shape[0] - 1
    D = w1_ref.shape[0]
    DK = D
    NC = k_scr.shape[0]
    CB = k_scr.shape[2]

    biases = jnp.dot(sins_ref[...], wk2_ref[...], preferred_element_type=f32)

    bf16 = jnp.bfloat16
    NL = DK // 128

    def slot_score(qb, kj):
        p = qb * kj
        s = p[:, :128]
        for l in range(1, NL):
            s = s + p[:, 128 * l:128 * (l + 1)]
        return jnp.sum(s, axis=-1, keepdims=True, dtype=f32)

    A = [None] * NC
    for c in range(NC):
        kv0 = jnp.dot(x_ref[c * CB:(c + 1) * CB], xw_ref[...],
                      preferred_element_type=f32)
        k_scr[c, 0] = kv0[:, :DK].astype(bf16)
        v_scr[c, 0] = kv0[:, DK:].astype(bf16)
        A[c] = kv0[:, DK:]

    for t in range(NI - 1):
        for c in range(NC):
            h = jnp.dot(A[c], w1_ref[...],
                        preferred_element_type=f32) + temb_ref[t]
            Bt = jnp.dot(jax.nn.gelu(h), w2_ref[...],
                         preferred_element_type=f32)
            kvq = jnp.dot(Bt, wkvq_ref[...], preferred_element_type=f32)
            k_scr[c, t + 1] = (kvq[:, :DK] + biases[t]).astype(bf16)
            v_scr[c, t + 1] = kvq[:, DK:DK + D].astype(bf16)
            qb = kvq[:, DK + D:].astype(bf16)

            n = t + 2
            svals = [slot_score(qb, k_scr[c, j]) for j in range(n)]
            m = svals[0]
            for j in range(1, n):
                m = jnp.maximum(m, svals[j])
            evals = [jnp.exp(s - m) for s in svals]
            den = evals[0]
            for j in range(1, n):
                den = den + evals[j]
            prods = [evals[j].astype(bf16) * v_scr[c, j] for j in range(n)]
            acc = None
            for j in range(0, n - 1, 2):
                pf = (prods[j] + prods[j + 1]).astype(f32)
                acc = pf if acc is None else acc + pf
            if n % 2:
                acc = acc + prods[n - 1].astype(f32)
            A[c] = acc * (1.0 / den)

    for c in range(NC):
        h = jnp.dot(A[c], w1_ref[...],
                    preferred_element_type=f32) + temb_ref[NI]
        o_ref[c * CB:(c + 1) * CB] = jnp.dot(
            jax.nn.gelu(h), w2_ref[...], preferred_element_type=f32)


@functools.partial(jax.jit, static_argnames=("interpret",))
def kernel(x, Wik, Wiv, Wq, Wk, Wv, W1, W2, t_emb, interpret=False):
    B, D = x.shape
    DK = Wq.shape[1]
    TR = Wk.shape[0] - D
    NI = t_emb.shape[0] - 1
    dt = x.dtype

    scale = np.float32(1.0 / np.sqrt(DK))
    wikp = jnp.pad(Wik, ((0, 0), (0, DK - Wik.shape[1])))
    xw = jnp.concatenate([wikp, Wiv], axis=1)
    wkvq = jnp.concatenate([Wk[:D], Wv, Wq * scale], axis=1)
    wk2 = Wk[D:]
    t_enc = jnp.pi * (0.5 ** jnp.arange(TR, dtype=dt))
    tvals = jnp.arange(NI, dtype=dt)
    sins = jnp.sin(tvals[:, None] * t_enc[None, :])

    BB = 256
    NC = 1
    CB = BB // NC
    grid = (B // BB,)
    full = lambda shape: pl.BlockSpec(shape, lambda i: tuple(0 for _ in shape))

    return pl.pallas_call(
        _kimia_body,
        out_shape=jax.ShapeDtypeStruct((B, D), dt),
        grid=grid,
        in_specs=[
            pl.BlockSpec((BB, D), lambda i: (i, 0)),
            full((D, DK + D)),
            full((D, DK + D + DK)),
            full((TR, DK)),
            full((D, D)),
            full((D, D)),
            full((NI + 1, D)),
            full((NI, TR)),
        ],
        out_specs=pl.BlockSpec((BB, D), lambda i: (i, 0)),
        scratch_shapes=[
            pltpu.VMEM((NC, NI, CB, DK), jnp.bfloat16),
            pltpu.VMEM((NC, NI, CB, D), jnp.bfloat16),
        ],
        compiler_params=pltpu.CompilerParams(
            dimension_semantics=("parallel",),
            vmem_limit_bytes=56 * 1024 * 1024,
        ),
        name="ind_kimia_fused",
        interpret=interpret,
    )(x, xw, wkvq, wk2, W1, W2, t_emb, sins)
